# Initial kernel scaffold; baseline (speedup 1.0000x reference)
import jax
import jax.numpy as jnp
from jax.experimental import pallas as pl


def kernel(x, edge_index, W1, b1, W2, b2):
    raise NotImplementedError("write your pallas kernel here")


# speedup vs baseline: 7.5555x; 7.5555x over previous
import functools

import jax
import jax.numpy as jnp
from jax import lax
from jax.experimental import pallas as pl
from jax.experimental.pallas import tpu as pltpu
from jax.experimental.pallas import tpu_sc as plsc

N = 10000
E = 320000
D = 128

NC = 2
NS = 16
NW = NC * NS
C = 64
K = 160
NBUF = 4
EPAD = NW * K * C
NPAD = 10240
RPT = NPAD // NS
DUMMY = N

ROWB = 1024
GRID = NPAD // ROWB


_MESH = plsc.VectorSubcoreMesh(core_axis_name="c", subcore_axis_name="s")


@functools.partial(
    pl.kernel,
    out_type=jax.ShapeDtypeStruct((NC, NPAD, D), jnp.float32),
    mesh=_MESH,
    scratch_types=(
        [pltpu.VMEM((C,), jnp.int32) for _ in range(NBUF)]
        + [pltpu.VMEM((C,), jnp.int32) for _ in range(NBUF)]
        + [pltpu.VMEM((C, D), jnp.float32) for _ in range(NBUF)]
        + [pltpu.VMEM_SHARED((NPAD, D), jnp.float32)]
        + [pltpu.SemaphoreType.DMA for _ in range(2 * NBUF)]
    ),
)
def _sc_message(src_hbm, dst_hbm, g_hbm, zeros_hbm, out_hbm, *refs):
    src_v = refs[:NBUF]
    dst_v = refs[NBUF:2 * NBUF]
    bufs = refs[2 * NBUF:3 * NBUF]
    acc = refs[3 * NBUF]
    isems = refs[3 * NBUF + 1:3 * NBUF + 1 + NBUF]
    gsems = refs[3 * NBUF + 1 + NBUF:]
    c = lax.axis_index("c")
    s = lax.axis_index("s")
    wid = c * NS + s
    pltpu.sync_copy(zeros_hbm, acc.at[pl.ds(s * RPT, RPT)])
    for b in range(NBUF):
        pltpu.make_async_copy(src_hbm.at[wid, b], src_v[b], isems[b]).start()
        pltpu.make_async_copy(dst_hbm.at[wid, b], dst_v[b], isems[b]).start()
    plsc.subcore_barrier()

    pltpu.make_async_copy(src_hbm.at[wid, 0], src_v[0], isems[0]).wait()
    pltpu.make_async_copy(dst_hbm.at[wid, 0], dst_v[0], isems[0]).wait()
    pltpu.make_async_copy(g_hbm.at[src_v[0]], bufs[0], gsems[0]).start()

    def step(j, carry):
        for b in range(NBUF):
            i = NBUF * j + b
            nb = (b + 1) % NBUF

            @pl.when(i + 1 < K)
            def _():
                pltpu.make_async_copy(src_hbm.at[wid, i + 1], src_v[nb],
                                      isems[nb]).wait()
                pltpu.make_async_copy(dst_hbm.at[wid, i + 1], dst_v[nb],
                                      isems[nb]).wait()
                pltpu.make_async_copy(g_hbm.at[src_v[nb]], bufs[nb],
                                      gsems[nb]).start()

            pltpu.make_async_copy(g_hbm.at[src_v[b]], bufs[b], gsems[b]).wait()
            pltpu.sync_copy(bufs[b], acc.at[dst_v[b]], add=True)

            @pl.when(i + NBUF < K)
            def _():
                pltpu.make_async_copy(src_hbm.at[wid, i + NBUF], src_v[b],
                                      isems[b]).start()
                pltpu.make_async_copy(dst_hbm.at[wid, i + NBUF], dst_v[b],
                                      isems[b]).start()

        return carry

    lax.fori_loop(0, K // NBUF, step, 0)
    plsc.subcore_barrier()
    pltpu.sync_copy(acc.at[pl.ds(s * RPT, RPT)],
                    out_hbm.at[c, pl.ds(s * RPT, RPT)])


def _row_mask(i):
    rows = lax.broadcasted_iota(jnp.int32, (ROWB, 1), 0) + i * ROWB
    return rows < N


def _dinv(degp_ref):
    dp = degp_ref[0] + degp_ref[1]
    deg = dp[:, 0:1] + 1.0
    return lax.rsqrt(deg)


def _tc1_body(x_ref, w_ref, degp_ref, g_ref):
    i = pl.program_id(0)
    h = jnp.dot(x_ref[...], w_ref[...], preferred_element_type=jnp.float32)
    g = h * _dinv(degp_ref)
    g_ref[...] = jnp.where(_row_mask(i), g, 0.0)


def _tc2_body(mp_ref, g1_ref, degp_ref, b_ref, w_ref, g2_ref):
    i = pl.program_id(0)
    dinv = _dinv(degp_ref)
    ssum = mp_ref[0] + mp_ref[1]
    pre = dinv * (ssum + g1_ref[...]) + b_ref[...]
    h = jnp.maximum(pre, 0.0)
    h2 = jnp.dot(h, w_ref[...], preferred_element_type=jnp.float32)
    g2_ref[...] = jnp.where(_row_mask(i), h2 * dinv, 0.0)


def _tc3_body(mp_ref, g2_ref, degp_ref, b_ref, out_ref):
    dinv = _dinv(degp_ref)
    ssum = mp_ref[0] + mp_ref[1]
    out_ref[...] = dinv * (ssum + g2_ref[...]) + b_ref[...]


_ROWS = pl.BlockSpec((ROWB, D), lambda i: (i, 0))
_FULLW = pl.BlockSpec((D, D), lambda i: (0, 0))
_MSGP = pl.BlockSpec((NC, ROWB, D), lambda i: (0, i, 0))
_BIAS = pl.BlockSpec((1, D), lambda i: (0, 0))

_tc1 = pl.pallas_call(
    _tc1_body,
    grid=(GRID,),
    in_specs=[_ROWS, _FULLW, _MSGP],
    out_specs=_ROWS,
    out_shape=jax.ShapeDtypeStruct((NPAD, D), jnp.float32),
)

_tc2 = pl.pallas_call(
    _tc2_body,
    grid=(GRID,),
    in_specs=[_MSGP, _ROWS, _MSGP, _BIAS, _FULLW],
    out_specs=_ROWS,
    out_shape=jax.ShapeDtypeStruct((NPAD, D), jnp.float32),
)

_tc3 = pl.pallas_call(
    _tc3_body,
    grid=(GRID,),
    in_specs=[_MSGP, _ROWS, _MSGP, _BIAS],
    out_specs=_ROWS,
    out_shape=jax.ShapeDtypeStruct((NPAD, D), jnp.float32),
)


def kernel(x, edge_index, W1, b1, W2, b2):
    src = edge_index[0].astype(jnp.int32)
    dst = edge_index[1].astype(jnp.int32)
    pad = jnp.full((EPAD - E,), DUMMY, dtype=jnp.int32)
    src_t = jnp.concatenate([src, pad]).reshape(NW, K, C)
    dst_t = jnp.concatenate([dst, pad]).reshape(NW, K, C)

    x_pad = jnp.pad(x, ((0, NPAD - N), (0, 0)))
    ones_table = jnp.ones((NPAD, D), jnp.float32)
    zerosD = jnp.zeros((RPT, D), jnp.float32)
    b1r = b1.reshape(1, D)
    b2r = b2.reshape(1, D)

    degp = _sc_message(src_t, dst_t, ones_table, zerosD)
    g1 = _tc1(x_pad, W1, degp)
    mp1 = _sc_message(src_t, dst_t, g1, zerosD)
    g2 = _tc2(mp1, g1, degp, b1r, W2)
    mp2 = _sc_message(src_t, dst_t, g2, zerosD)
    out = _tc3(mp2, g2, degp, b2r)
    return out[:N]


# speedup vs baseline: 7.6558x; 1.0133x over previous
import functools

import jax
import jax.numpy as jnp
from jax import lax
from jax.experimental import pallas as pl
from jax.experimental.pallas import tpu as pltpu
from jax.experimental.pallas import tpu_sc as plsc

N = 10000
E = 320000
D = 128

NC = 2
NS = 16
NW = NC * NS
C = 64
K = 160
NBUF = 4
NI = 8
EPAD = NW * K * C
NPAD = 10240
RPT = NPAD // NS
DUMMY = N

ROWB = 1024
GRID = NPAD // ROWB


_MESH = plsc.VectorSubcoreMesh(core_axis_name="c", subcore_axis_name="s")


@functools.partial(
    pl.kernel,
    out_type=jax.ShapeDtypeStruct((NC, NPAD, D), jnp.float32),
    mesh=_MESH,
    scratch_types=(
        [pltpu.VMEM((C,), jnp.int32) for _ in range(NI)]
        + [pltpu.VMEM((C,), jnp.int32) for _ in range(NI)]
        + [pltpu.VMEM((C, D), jnp.float32) for _ in range(NBUF)]
        + [pltpu.VMEM_SHARED((NPAD, D), jnp.float32)]
        + [pltpu.SemaphoreType.DMA for _ in range(NI)]
        + [pltpu.SemaphoreType.DMA for _ in range(NBUF)]
        + [pltpu.SemaphoreType.DMA for _ in range(NBUF)]
    ),
)
def _sc_message(src_hbm, dst_hbm, g_hbm, zeros_hbm, out_hbm, *refs):
    src_v = refs[:NI]
    dst_v = refs[NI:2 * NI]
    bufs = refs[2 * NI:2 * NI + NBUF]
    acc = refs[2 * NI + NBUF]
    o = 2 * NI + NBUF + 1
    isems = refs[o:o + NI]
    gsems = refs[o + NI:o + NI + NBUF]
    ssems = refs[o + NI + NBUF:]
    c = lax.axis_index("c")
    s = lax.axis_index("s")
    wid = c * NS + s

    def idx_start(i, sl):
        pltpu.make_async_copy(src_hbm.at[wid, i], src_v[sl], isems[sl]).start()
        pltpu.make_async_copy(dst_hbm.at[wid, i], dst_v[sl], isems[sl]).start()

    def idx_wait(i, sl):
        pltpu.make_async_copy(src_hbm.at[wid, i], src_v[sl], isems[sl]).wait()
        pltpu.make_async_copy(dst_hbm.at[wid, i], dst_v[sl], isems[sl]).wait()

    def gather(sl8, sl4):
        return pltpu.make_async_copy(g_hbm.at[src_v[sl8]], bufs[sl4],
                                     gsems[sl4])

    def scatter(sl8, sl4):
        return pltpu.make_async_copy(bufs[sl4], acc.at[dst_v[sl8]],
                                     ssems[sl4])

    pltpu.sync_copy(zeros_hbm, acc.at[pl.ds(s * RPT, RPT)])
    for f in range(NI - 2):
        idx_start(f, f)
    idx_wait(0, 0)
    gather(0, 0).start()
    idx_wait(1, 1)
    gather(1, 1).start()
    plsc.subcore_barrier()

    def step(jj, carry):
        for t in range(NI):
            j = NI * jj + t

            @pl.when(j >= 2)
            def _():
                scatter((t + 6) % NI, (t + 2) % NBUF).wait()

            @pl.when(j + 6 < K)
            def _():
                idx_start(j + 6, (t + 6) % NI)

            @pl.when(j + 2 < K)
            def _():
                idx_wait(j + 2, (t + 2) % NI)
                gather((t + 2) % NI, (t + 2) % NBUF).start()

            gather(t % NI, t % NBUF).wait()
            scatter(t % NI, t % NBUF).start(add=True)

        return carry

    lax.fori_loop(0, K // NI, step, 0)
    scatter((K - 2) % NI, (K - 2) % NBUF).wait()
    scatter((K - 1) % NI, (K - 1) % NBUF).wait()
    plsc.subcore_barrier()
    pltpu.sync_copy(acc.at[pl.ds(s * RPT, RPT)],
                    out_hbm.at[c, pl.ds(s * RPT, RPT)])


def _row_mask(i):
    rows = lax.broadcasted_iota(jnp.int32, (ROWB, 1), 0) + i * ROWB
    return rows < N


def _dinv(degp_ref):
    dp = degp_ref[0] + degp_ref[1]
    deg = dp[:, 0:1] + 1.0
    return lax.rsqrt(deg)


def _tc1_body(x_ref, w_ref, degp_ref, g_ref):
    i = pl.program_id(0)
    h = jnp.dot(x_ref[...], w_ref[...], preferred_element_type=jnp.float32)
    g = h * _dinv(degp_ref)
    g_ref[...] = jnp.where(_row_mask(i), g, 0.0)


def _tc2_body(mp_ref, g1_ref, degp_ref, b_ref, w_ref, g2_ref):
    i = pl.program_id(0)
    dinv = _dinv(degp_ref)
    ssum = mp_ref[0] + mp_ref[1]
    pre = dinv * (ssum + g1_ref[...]) + b_ref[...]
    h = jnp.maximum(pre, 0.0)
    h2 = jnp.dot(h, w_ref[...], preferred_element_type=jnp.float32)
    g2_ref[...] = jnp.where(_row_mask(i), h2 * dinv, 0.0)


def _tc3_body(mp_ref, g2_ref, degp_ref, b_ref, out_ref):
    dinv = _dinv(degp_ref)
    ssum = mp_ref[0] + mp_ref[1]
    out_ref[...] = dinv * (ssum + g2_ref[...]) + b_ref[...]


_ROWS = pl.BlockSpec((ROWB, D), lambda i: (i, 0))
_FULLW = pl.BlockSpec((D, D), lambda i: (0, 0))
_MSGP = pl.BlockSpec((NC, ROWB, D), lambda i: (0, i, 0))
_BIAS = pl.BlockSpec((1, D), lambda i: (0, 0))

_tc1 = pl.pallas_call(
    _tc1_body,
    grid=(GRID,),
    in_specs=[_ROWS, _FULLW, _MSGP],
    out_specs=_ROWS,
    out_shape=jax.ShapeDtypeStruct((NPAD, D), jnp.float32),
)

_tc2 = pl.pallas_call(
    _tc2_body,
    grid=(GRID,),
    in_specs=[_MSGP, _ROWS, _MSGP, _BIAS, _FULLW],
    out_specs=_ROWS,
    out_shape=jax.ShapeDtypeStruct((NPAD, D), jnp.float32),
)

_tc3 = pl.pallas_call(
    _tc3_body,
    grid=(GRID,),
    in_specs=[_MSGP, _ROWS, _MSGP, _BIAS],
    out_specs=_ROWS,
    out_shape=jax.ShapeDtypeStruct((NPAD, D), jnp.float32),
)


def kernel(x, edge_index, W1, b1, W2, b2):
    src = edge_index[0].astype(jnp.int32)
    dst = edge_index[1].astype(jnp.int32)
    pad = jnp.full((EPAD - E,), DUMMY, dtype=jnp.int32)
    src_t = jnp.concatenate([src, pad]).reshape(NW, K, C)
    dst_t = jnp.concatenate([dst, pad]).reshape(NW, K, C)

    x_pad = jnp.pad(x, ((0, NPAD - N), (0, 0)))
    ones_table = jnp.ones((NPAD, D), jnp.float32)
    zerosD = jnp.zeros((RPT, D), jnp.float32)
    b1r = b1.reshape(1, D)
    b2r = b2.reshape(1, D)

    degp = _sc_message(src_t, dst_t, ones_table, zerosD)
    g1 = _tc1(x_pad, W1, degp)
    mp1 = _sc_message(src_t, dst_t, g1, zerosD)
    g2 = _tc2(mp1, g1, degp, b1r, W2)
    mp2 = _sc_message(src_t, dst_t, g2, zerosD)
    out = _tc3(mp2, g2, degp, b2r)
    return out[:N]
